# baseline (speedup 1.0000x reference)
import functools
import jax
import jax.numpy as jnp
from jax.experimental import pallas as pl
from jax.experimental.pallas import tpu as pltpu

N = 10000
E = 320000
D = 128
KS = (5000, 2500)


def _mm_body(agg_ref, ni_ref, w_ref, b_ref, o_ref, *, relu):
    a = agg_ref[...] * ni_ref[...]
    out = jnp.dot(a, w_ref[...], preferred_element_type=jnp.float32) + b_ref[...]
    if relu:
        out = jnp.maximum(out, 0.0)
    o_ref[...] = out


def _gcn_mm(agg, ni, W, b, relu):
    n = agg.shape[0]
    blk = 400
    grid = pl.cdiv(n, blk)
    return pl.pallas_call(
        functools.partial(_mm_body, relu=relu),
        grid=(grid,),
        in_specs=[
            pl.BlockSpec((blk, D), lambda i: (i, 0)),
            pl.BlockSpec((blk, 1), lambda i: (i, 0)),
            pl.BlockSpec((D, D), lambda i: (0, 0)),
            pl.BlockSpec((1, D), lambda i: (0, 0)),
        ],
        out_specs=pl.BlockSpec((blk, D), lambda i: (i, 0)),
        out_shape=jax.ShapeDtypeStruct((n, D), jnp.float32),
    )(agg, ni.reshape(n, 1), W, b.reshape(1, D))


def _gcn(src, dst, emask, n, h, W, b, relu):
    deg_o = jnp.zeros((n,), jnp.float32).at[src].add(emask)
    deg_i = jnp.zeros((n,), jnp.float32).at[dst].add(emask)
    no = jax.lax.rsqrt(jnp.maximum(deg_o, 1.0))
    ni = jax.lax.rsqrt(jnp.maximum(deg_i, 1.0))
    msg = (h * no[:, None])[src] * emask[:, None]
    agg = jnp.zeros((n, h.shape[1]), jnp.float32).at[dst].add(msg)
    return _gcn_mm(agg, ni, W, b, relu)


def _pool_edges(src, dst, emask, nids, n_old, k):
    mapping = jnp.full((n_old,), -1, jnp.int32).at[nids].set(
        jnp.arange(k, dtype=jnp.int32))
    ms = mapping[src]
    md = mapping[dst]
    keep = ((ms >= 0) & (md >= 0)).astype(jnp.float32) * emask
    return jnp.where(ms >= 0, ms, 0), jnp.where(md >= 0, md, 0), keep


def kernel(x, edge_index, We, be, W0, b0, W1, b1, Wb, bb, Wd0, bd0, Wd1, bd1,
           Wp0, bp0, Wp1, bp1):
    src = edge_index[0]
    dst = edge_index[1]
    e0 = jnp.ones((E,), jnp.float32)
    h = _gcn(src, dst, e0, N, x, We, be, True)
    h = _gcn(src, dst, e0, N, h, W0, b0, True)
    hid0 = h
    s0 = jax.nn.sigmoid(h @ Wp0 + bp0)
    _, nids0 = jax.lax.top_k(s0[:, 0], KS[0])
    src1, dst1, e1 = _pool_edges(src, dst, e0, nids0, N, KS[0])
    h = h[nids0] * s0[nids0]
    h = _gcn(src1, dst1, e1, KS[0], h, W1, b1, True)
    hid1 = h
    s1 = jax.nn.sigmoid(h @ Wp1 + bp1)
    _, nids1 = jax.lax.top_k(s1[:, 0], KS[1])
    src2, dst2, e2 = _pool_edges(src1, dst1, e1, nids1, KS[0], KS[1])
    h = h[nids1] * s1[nids1]
    h = _gcn(src2, dst2, e2, KS[1], h, Wb, bb, True)
    h = jnp.zeros((KS[0], D), jnp.float32).at[nids1].set(h)
    h = h + hid1
    h = _gcn(src1, dst1, e1, KS[0], h, Wd0, bd0, True)
    h = jnp.zeros((N, D), jnp.float32).at[nids0].set(h)
    h = h + hid0
    h = _gcn(src, dst, e0, N, h, Wd1, bd1, False)
    return h


# speedup vs baseline: 1.8860x; 1.8860x over previous
import functools
import jax
import jax.numpy as jnp
from jax import lax
from jax.experimental import pallas as pl
from jax.experimental.pallas import tpu as pltpu
from jax.experimental.pallas import tpu_sc as plsc

N = 10000
E = 320000
D = 128
KS = (5000, 2500)

NC = 2
NS = 16
NW = NC * NS
CH = 128
NCHK = E // CH

PAD1 = 136
PAD2 = 140

_MESH = plsc.VectorSubcoreMesh(core_axis_name="c", subcore_axis_name="s")


def _mp_body(tab, srcf, dstf, zer, out, acc, sva, dva, svb, dvb,
             rows_a, rows_b, sga, sgb, ssa, ssb, *, n_acc):
    cid = lax.axis_index("c")
    sid = lax.axis_index("s")
    wid = sid * NC + cid
    rpt = n_acc // NS
    pltpu.sync_copy(zer.at[pl.ds(sid * rpt, rpt)], acc.at[pl.ds(sid * rpt, rpt)])
    plsc.subcore_barrier()

    def do_chunk(c, idx_s, idx_d, rows, sg, ss):
        pltpu.sync_copy(srcf.at[pl.ds(c * CH, CH)], idx_s)
        g = pltpu.async_copy(tab.at[idx_s], rows, sg)
        pltpu.sync_copy(dstf.at[pl.ds(c * CH, CH)], idx_d)
        g.wait()
        return pltpu.async_copy(rows, acc.at[idx_d], ss, add=True)

    def chunk_pair(i, _):
        c0 = wid + (2 * i) * NW
        sa = do_chunk(c0, sva, dva, rows_a, sga, ssa)
        sb = do_chunk(c0 + NW, svb, dvb, rows_b, sgb, ssb)
        sa.wait()
        sb.wait()
        return 0

    lax.fori_loop(0, (NCHK // NW) // 2, chunk_pair, 0)

    @pl.when(wid < NCHK % NW)
    def _():
        do_chunk(wid + (NCHK // NW) * NW, sva, dva, rows_a, sga, ssa).wait()

    plsc.subcore_barrier()
    pltpu.sync_copy(acc.at[pl.ds(sid * rpt, rpt)],
                    out.at[cid, pl.ds(sid * rpt, rpt)])


def _make_mp(n_tab, n_acc):
    body = functools.partial(_mp_body, n_acc=n_acc)
    return pl.kernel(
        body,
        out_type=jax.ShapeDtypeStruct((NC, n_acc, D), jnp.float32),
        mesh=_MESH,
        scratch_types=[
            pltpu.VMEM_SHARED((n_acc, D), jnp.float32),
            pltpu.VMEM((CH,), jnp.int32),
            pltpu.VMEM((CH,), jnp.int32),
            pltpu.VMEM((CH,), jnp.int32),
            pltpu.VMEM((CH,), jnp.int32),
            pltpu.VMEM((CH, D), jnp.float32),
            pltpu.VMEM((CH, D), jnp.float32),
            pltpu.SemaphoreType.DMA,
            pltpu.SemaphoreType.DMA,
            pltpu.SemaphoreType.DMA,
            pltpu.SemaphoreType.DMA,
        ],
    )


RW = 128


def _deg_body(srcf, dstf, onesr_h, zd_h, out, acc, idx_v, ones_v, sem,
              *, n_deg):
    cid = lax.axis_index("c")
    sid = lax.axis_index("s")
    rpt = n_deg // NS
    pltpu.sync_copy(onesr_h, ones_v)
    pltpu.sync_copy(zd_h.at[pl.ds(sid * rpt, rpt)],
                    acc.at[pl.ds(sid * rpt, rpt)])
    plsc.subcore_barrier()

    def scan_all(ef):
        def chunk(g, _):
            c = sid + g * NS
            pltpu.sync_copy(ef.at[pl.ds(c * CH, CH)], idx_v)
            pltpu.sync_copy(ones_v, acc.at[idx_v], add=True)
            return 0

        lax.fori_loop(0, NCHK // NS, chunk, 0)

        @pl.when(sid < NCHK % NS)
        def _():
            chunk(NCHK // NS, 0)

    @pl.when(cid == 0)
    def _():
        scan_all(srcf)

    @pl.when(cid == 1)
    def _():
        scan_all(dstf)

    plsc.subcore_barrier()
    pltpu.sync_copy(acc.at[pl.ds(sid * rpt, rpt)],
                    out.at[cid, pl.ds(sid * rpt, rpt)])


def _mp_jax(n_acc):
    def f(tab, srcf, dstf, zer):
        agg = jnp.zeros((n_acc, D), jnp.float32).at[dstf].add(tab[srcf])
        return jnp.stack([agg, jnp.zeros_like(agg)])
    return f

def _make_deg(n_deg):
    body = functools.partial(_deg_body, n_deg=n_deg)
    return pl.kernel(
        body,
        out_type=jax.ShapeDtypeStruct((NC, n_deg, RW), jnp.float32),
        mesh=_MESH,
        scratch_types=[
            pltpu.VMEM_SHARED((n_deg, RW), jnp.float32),
            pltpu.VMEM((CH,), jnp.int32),
            pltpu.VMEM((CH, RW), jnp.float32),
            pltpu.SemaphoreType.DMA,
        ],
    )


def _layer_body(p0, p1, di, w, b, o, *, relu):
    ni = lax.rsqrt(jnp.maximum(di[...], 1.0))
    agg = (p0[...] + p1[...]) * ni
    out = jnp.dot(agg, w[...], preferred_element_type=jnp.float32) + b[...]
    if relu:
        out = jnp.maximum(out, 0.0)
    o[...] = out


def _layer_proj_body(p0, p1, di, w, b, wp, bp, o, s, *, relu):
    ni = lax.rsqrt(jnp.maximum(di[...], 1.0))
    agg = (p0[...] + p1[...]) * ni
    out = jnp.dot(agg, w[...], preferred_element_type=jnp.float32) + b[...]
    if relu:
        out = jnp.maximum(out, 0.0)
    o[...] = out
    s[...] = jax.nn.sigmoid(
        jnp.dot(out, wp[...], preferred_element_type=jnp.float32) + bp[...])


_BLK = 512


def _row_spec(blk=_BLK, dim=D):
    return pl.BlockSpec((blk, dim), lambda i: (i, 0))


def _one_spec(dim):
    return pl.BlockSpec((1, dim) if dim > 1 else (1, 1), lambda i: (0, 0))


def _tc_layer(p, di, W, b, relu, n, proj=None):
    grid = (pl.cdiv(n, _BLK),)
    p0, p1 = p[0], p[1]
    div = di.reshape(-1, 1)[:n]
    in_specs = [_row_spec(), _row_spec(), _row_spec(dim=1),
                pl.BlockSpec((D, D), lambda i: (0, 0)), _one_spec(D)]
    args = [p0, p1, div, W, b.reshape(1, D)]
    if proj is None:
        return pl.pallas_call(
            functools.partial(_layer_body, relu=relu),
            grid=grid,
            in_specs=in_specs,
            out_specs=_row_spec(),
            out_shape=jax.ShapeDtypeStruct((n, D), jnp.float32),
        )(*args)
    Wp, bp = proj
    return pl.pallas_call(
        functools.partial(_layer_proj_body, relu=relu),
        grid=grid,
        in_specs=in_specs + [pl.BlockSpec((D, 1), lambda i: (0, 0)),
                             _one_spec(1)],
        out_specs=[_row_spec(), _row_spec(dim=1)],
        out_shape=[jax.ShapeDtypeStruct((n, D), jnp.float32),
                   jax.ShapeDtypeStruct((n, 1), jnp.float32)],
    )(*args, Wp, bp.reshape(1, 1))


def _scale_body(x, do, s, o, *, n_valid, blk):
    i = pl.program_id(0)
    no = lax.rsqrt(jnp.maximum(do[...], 1.0))
    out = x[...] * no
    if s is not None:
        out = out * s[...]
    rows = i * blk + lax.broadcasted_iota(jnp.int32, (blk, 1), 0)
    o[...] = jnp.where(rows < n_valid, out, 0.0)


def _tc_scale(x, do, s, n_out):
    n = x.shape[0]
    grid = (pl.cdiv(n_out, _BLK),)
    dov = do.reshape(-1, 1)[:n]
    nbx = pl.cdiv(n, _BLK)

    def xmap(i):
        return (jnp.minimum(i, nbx - 1), 0)

    xspec = pl.BlockSpec((_BLK, D), xmap)
    dspec = pl.BlockSpec((_BLK, 1), xmap)
    if s is None:
        def body2(x_r, d_r, o_r):
            _scale_body(x_r, d_r, None, o_r, n_valid=n, blk=_BLK)
        return pl.pallas_call(
            body2, grid=grid,
            in_specs=[xspec, dspec],
            out_specs=_row_spec(),
            out_shape=jax.ShapeDtypeStruct((n_out, D), jnp.float32),
        )(x, dov)

    def body(x_r, d_r, s_r, o_r):
        _scale_body(x_r, d_r, s_r, o_r, n_valid=n, blk=_BLK)
    return pl.pallas_call(
        body, grid=grid,
        in_specs=[xspec, dspec, dspec],
        out_specs=_row_spec(),
        out_shape=jax.ShapeDtypeStruct((n_out, D), jnp.float32),
    )(x, dov, s.reshape(-1, 1))


def _relabel(msrc, mdst, n_new, pad, eids):
    keep = (msrc >= 0) & (mdst >= 0)
    dummy = n_new + eids % pad
    return (jnp.where(keep, msrc, dummy).astype(jnp.int32),
            jnp.where(keep, mdst, dummy).astype(jnp.int32))


def kernel(x, edge_index, We, be, W0, b0, W1, b1, Wb, bb, Wd0, bd0, Wd1, bd1,
           Wp0, bp0, Wp1, bp1):
    src = edge_index[0]
    dst = edge_index[1]
    eids = jnp.arange(E, dtype=jnp.int32)

    NDF, ND1, ND2 = 10240, 5248, 2688
    zN = jnp.zeros((NDF, D), jnp.float32)
    z1 = jnp.zeros((ND1, D), jnp.float32)
    z2 = jnp.zeros((ND2, D), jnp.float32)
    zdF = jnp.zeros((NDF, RW), jnp.float32)
    zd1 = jnp.zeros((ND1, RW), jnp.float32)
    zd2 = jnp.zeros((ND2, RW), jnp.float32)
    onesr = jnp.zeros((CH, RW), jnp.float32).at[:, 0].set(1.0)

    mpF = _make_mp(N, NDF)
    mp1 = _make_mp(KS[0] + PAD1, ND1)
    mp2 = _make_mp(KS[1] + PAD2, ND2)
    degF = _make_deg(NDF)
    deg1 = _make_deg(ND1)
    deg2 = _make_deg(ND2)

    dF = degF(src, dst, onesr, zdF)
    doF = dF[0, :, 0]
    diF = dF[1, :, 0]

    xs = _tc_scale(x, doF, None, N)
    p = mpF(xs, src, dst, zN)
    hs = _tc_layer((p[0], p[1]), diF, We, be, True, N)
    hs = _tc_scale(hs, doF, None, N)
    p = mpF(hs, src, dst, zN)
    hid0, s0 = _tc_layer((p[0], p[1]), diF, W0, b0, True, N,
                         proj=(Wp0, bp0))

    _, nids0 = lax.top_k(s0[:, 0], KS[0])
    nids0 = nids0.astype(jnp.int32)
    map0 = jnp.full((N,), -1, jnp.int32).at[nids0].set(
        jnp.arange(KS[0], dtype=jnp.int32))
    sl1, dl1 = _relabel(map0[src], map0[dst], KS[0], PAD1, eids)
    d1 = deg1(sl1, dl1, onesr, zd1)
    do1 = d1[0, :, 0]
    di1 = d1[1, :, 0]

    h1 = hid0[nids0]
    sg0 = s0[nids0, 0]
    t1 = _tc_scale(h1, do1, sg0, KS[0] + PAD1)
    p = mp1(t1, sl1, dl1, z1)
    hid1, s1 = _tc_layer((p[0], p[1]), di1, W1, b1, True, KS[0],
                         proj=(Wp1, bp1))

    _, nids1 = lax.top_k(s1[:, 0], KS[1])
    nids1 = nids1.astype(jnp.int32)
    map1 = jnp.full((KS[0] + PAD1,), -1, jnp.int32).at[nids1].set(
        jnp.arange(KS[1], dtype=jnp.int32))
    sl2, dl2 = _relabel(map1[sl1], map1[dl1], KS[1], PAD2, eids)
    d2 = deg2(sl2, dl2, onesr, zd2)
    do2 = d2[0, :, 0]
    di2 = d2[1, :, 0]

    h2 = hid1[nids1]
    sg1 = s1[nids1, 0]
    t2 = _tc_scale(h2, do2, sg1, KS[1] + PAD2)
    p = mp2(t2, sl2, dl2, z2)
    hb = _tc_layer((p[0], p[1]), di2, Wb, bb, True, KS[1])

    u1 = hid1.at[nids1].add(hb)
    td0 = _tc_scale(u1, do1, None, KS[0] + PAD1)
    p = mp1(td0, sl1, dl1, z1)
    hd0 = _tc_layer((p[0], p[1]), di1, Wd0, bd0, True, KS[0])

    u0 = hid0.at[nids0].add(hd0)
    td1 = _tc_scale(u0, doF, None, N)
    p = mpF(td1, src, dst, zN)
    return _tc_layer((p[0], p[1]), diF, Wd1, bd1, False, N)


# speedup vs baseline: 8.0330x; 4.2592x over previous
import functools
import jax
import jax.numpy as jnp
from jax import lax
from jax.experimental import pallas as pl
from jax.experimental.pallas import tpu as pltpu
from jax.experimental.pallas import tpu_sc as plsc

N = 10000
E = 320000
D = 128
KS = (5000, 2500)

NC = 2
NS = 16
NW = NC * NS
CH = 128
NCHK = E // CH

PAD1 = 136
PAD2 = 140

_MESH = plsc.VectorSubcoreMesh(core_axis_name="c", subcore_axis_name="s")


def _mp_body(tab, srcf, dstf, zer, out, acc, sva, dva, svb, dvb,
             rows_a, rows_b, sga, sgb, ssa, ssb, *, n_acc):
    cid = lax.axis_index("c")
    sid = lax.axis_index("s")
    wid = sid * NC + cid
    rpt = n_acc // NS
    pltpu.sync_copy(zer.at[pl.ds(sid * rpt, rpt)], acc.at[pl.ds(sid * rpt, rpt)])
    plsc.subcore_barrier()

    def do_chunk(c, idx_s, idx_d, rows, sg, ss):
        pltpu.sync_copy(srcf.at[pl.ds(c * CH, CH)], idx_s)
        g = pltpu.async_copy(tab.at[idx_s], rows, sg)
        pltpu.sync_copy(dstf.at[pl.ds(c * CH, CH)], idx_d)
        g.wait()
        return pltpu.async_copy(rows, acc.at[idx_d], ss, add=True)

    def chunk_pair(i, _):
        c0 = wid + (2 * i) * NW
        sa = do_chunk(c0, sva, dva, rows_a, sga, ssa)
        sb = do_chunk(c0 + NW, svb, dvb, rows_b, sgb, ssb)
        sa.wait()
        sb.wait()
        return 0

    lax.fori_loop(0, (NCHK // NW) // 2, chunk_pair, 0)

    @pl.when(wid < NCHK % NW)
    def _():
        do_chunk(wid + (NCHK // NW) * NW, sva, dva, rows_a, sga, ssa).wait()

    plsc.subcore_barrier()
    pltpu.sync_copy(acc.at[pl.ds(sid * rpt, rpt)],
                    out.at[cid, pl.ds(sid * rpt, rpt)])


def _make_mp(n_tab, n_acc):
    body = functools.partial(_mp_body, n_acc=n_acc)
    return pl.kernel(
        body,
        out_type=jax.ShapeDtypeStruct((NC, n_acc, D), jnp.float32),
        mesh=_MESH,
        scratch_types=[
            pltpu.VMEM_SHARED((n_acc, D), jnp.float32),
            pltpu.VMEM((CH,), jnp.int32),
            pltpu.VMEM((CH,), jnp.int32),
            pltpu.VMEM((CH,), jnp.int32),
            pltpu.VMEM((CH,), jnp.int32),
            pltpu.VMEM((CH, D), jnp.float32),
            pltpu.VMEM((CH, D), jnp.float32),
            pltpu.SemaphoreType.DMA,
            pltpu.SemaphoreType.DMA,
            pltpu.SemaphoreType.DMA,
            pltpu.SemaphoreType.DMA,
        ],
    )


RW = 128


def _deg_body(srcf, dstf, onesr_h, zd_h, out, acc, idx_v, ones_v, sem,
              *, n_deg):
    cid = lax.axis_index("c")
    sid = lax.axis_index("s")
    rpt = n_deg // NS
    pltpu.sync_copy(onesr_h, ones_v)
    pltpu.sync_copy(zd_h.at[pl.ds(sid * rpt, rpt)],
                    acc.at[pl.ds(sid * rpt, rpt)])
    plsc.subcore_barrier()

    def scan_all(ef):
        def chunk(g, _):
            c = sid + g * NS
            pltpu.sync_copy(ef.at[pl.ds(c * CH, CH)], idx_v)
            pltpu.sync_copy(ones_v, acc.at[idx_v], add=True)
            return 0

        lax.fori_loop(0, NCHK // NS, chunk, 0)

        @pl.when(sid < NCHK % NS)
        def _():
            chunk(NCHK // NS, 0)

    @pl.when(cid == 0)
    def _():
        scan_all(srcf)

    @pl.when(cid == 1)
    def _():
        scan_all(dstf)

    plsc.subcore_barrier()
    pltpu.sync_copy(acc.at[pl.ds(sid * rpt, rpt)],
                    out.at[cid, pl.ds(sid * rpt, rpt)])


def _make_deg(n_deg):
    body = functools.partial(_deg_body, n_deg=n_deg)
    return pl.kernel(
        body,
        out_type=jax.ShapeDtypeStruct((NC, n_deg, RW), jnp.float32),
        mesh=_MESH,
        scratch_types=[
            pltpu.VMEM_SHARED((n_deg, RW), jnp.float32),
            pltpu.VMEM((CH,), jnp.int32),
            pltpu.VMEM((CH, RW), jnp.float32),
            pltpu.SemaphoreType.DMA,
        ],
    )


def _degmp_body(ind, srcf, dstf, zer, out, acc, sva, dva, svb, dvb,
                rows_a, rows_b, sga, sgb, ssa, ssb, *, n_acc):
    cid = lax.axis_index("c")
    sid = lax.axis_index("s")
    rpt = n_acc // NS
    pltpu.sync_copy(zer.at[pl.ds(sid * rpt, rpt)], acc.at[pl.ds(sid * rpt, rpt)])
    plsc.subcore_barrier()

    def scan(gf, sf):
        def do_chunk(c, idx_g, idx_s, rows, sg, ss):
            pltpu.sync_copy(gf.at[pl.ds(c * CH, CH)], idx_g)
            g = pltpu.async_copy(ind.at[idx_g], rows, sg)
            pltpu.sync_copy(sf.at[pl.ds(c * CH, CH)], idx_s)
            g.wait()
            return pltpu.async_copy(rows, acc.at[idx_s], ss, add=True)

        def chunk_pair(i, _):
            c0 = sid + (2 * i) * NS
            sa = do_chunk(c0, sva, dva, rows_a, sga, ssa)
            sb = do_chunk(c0 + NS, svb, dvb, rows_b, sgb, ssb)
            sa.wait()
            sb.wait()
            return 0

        lax.fori_loop(0, (NCHK // NS) // 2, chunk_pair, 0)

        @pl.when(sid < NCHK % NS)
        def _():
            do_chunk(sid + (NCHK // NS) * NS, sva, dva, rows_a, sga, ssa).wait()

    @pl.when(cid == 0)
    def _():
        scan(dstf, srcf)

    @pl.when(cid == 1)
    def _():
        scan(srcf, dstf)

    plsc.subcore_barrier()
    pltpu.sync_copy(acc.at[pl.ds(sid * rpt, rpt)],
                    out.at[cid, pl.ds(sid * rpt, rpt)])


def _make_degmp(n_acc):
    body = functools.partial(_degmp_body, n_acc=n_acc)
    return pl.kernel(
        body,
        out_type=jax.ShapeDtypeStruct((NC, n_acc, D), jnp.float32),
        mesh=_MESH,
        scratch_types=[
            pltpu.VMEM_SHARED((n_acc, D), jnp.float32),
            pltpu.VMEM((CH,), jnp.int32),
            pltpu.VMEM((CH,), jnp.int32),
            pltpu.VMEM((CH,), jnp.int32),
            pltpu.VMEM((CH,), jnp.int32),
            pltpu.VMEM((CH, D), jnp.float32),
            pltpu.VMEM((CH, D), jnp.float32),
            pltpu.SemaphoreType.DMA,
            pltpu.SemaphoreType.DMA,
            pltpu.SemaphoreType.DMA,
            pltpu.SemaphoreType.DMA,
        ],
    )


def _layer_body(p0, p1, di, w, b, sel, o, *, relu):
    ni = lax.rsqrt(jnp.maximum(di[...], 1.0))
    agg = (p0[...] + p1[...]) * ni
    out = jnp.dot(agg, w[...], preferred_element_type=jnp.float32) + b[...]
    if relu:
        out = jnp.maximum(out, 0.0)
    if sel is not None:
        out = out * sel[...]
    o[...] = out


def _layer_proj_body(p0, p1, di, w, b, sel, wp, bp, o, sc, *, relu):
    ni = lax.rsqrt(jnp.maximum(di[...], 1.0))
    agg = (p0[...] + p1[...]) * ni
    out = jnp.dot(agg, w[...], preferred_element_type=jnp.float32) + b[...]
    if relu:
        out = jnp.maximum(out, 0.0)
    if sel is not None:
        out = out * sel[...]
    o[...] = out
    sv = jax.nn.sigmoid(
        jnp.dot(out, wp[...], preferred_element_type=jnp.float32) + bp[...])
    if sel is not None:
        sv = jnp.where(sel[...] > 0.0, sv, -1e30)
    sc[...] = sv


_BLK = 512


def _row_spec(blk=_BLK, dim=D):
    return pl.BlockSpec((blk, dim), lambda i: (i, 0))


def _one_spec(dim):
    return pl.BlockSpec((1, dim) if dim > 1 else (1, 1), lambda i: (0, 0))


def _tc_layer(p, di, W, b, relu, sel=None, proj=None):
    grid = (pl.cdiv(N, _BLK),)
    div = di.reshape(-1, 1)[:N]
    in_specs = [_row_spec(), _row_spec(), _row_spec(dim=1),
                pl.BlockSpec((D, D), lambda i: (0, 0)), _one_spec(D)]
    args = [p[0], p[1], div, W, b.reshape(1, D)]
    if sel is not None:
        in_specs.append(_row_spec(dim=1))
        args.append(sel.reshape(N, 1))
    else:
        in_specs.append(None)
        args.append(None)
    sspec = [sp for sp in in_specs if sp is not None]
    sargs = [a for a in args if a is not None]
    has_sel = sel is not None
    if proj is None:
        def body(*refs):
            if has_sel:
                p0, p1, dr, w, b_, sl, o = refs
            else:
                (p0, p1, dr, w, b_, o), sl = refs, None
            _layer_body(p0, p1, dr, w, b_, sl, o, relu=relu)
        return pl.pallas_call(
            body, grid=grid, in_specs=sspec,
            out_specs=_row_spec(),
            out_shape=jax.ShapeDtypeStruct((N, D), jnp.float32),
        )(*sargs)
    Wp, bp = proj

    def bodyp(*refs):
        if has_sel:
            p0, p1, dr, w, b_, sl, wp, bp_, o, sc = refs
        else:
            (p0, p1, dr, w, b_, wp, bp_, o, sc), sl = refs, None
        _layer_proj_body(p0, p1, dr, w, b_, sl, wp, bp_, o, sc, relu=relu)
    return pl.pallas_call(
        bodyp, grid=grid,
        in_specs=sspec + [pl.BlockSpec((D, 1), lambda i: (0, 0)),
                          _one_spec(1)],
        out_specs=[_row_spec(), _row_spec(dim=1)],
        out_shape=[jax.ShapeDtypeStruct((N, D), jnp.float32),
                   jax.ShapeDtypeStruct((N, 1), jnp.float32)],
    )(*sargs, Wp, bp.reshape(1, 1))


def _tc_scale(x, do, s=None, sel=None, addto=None):
    grid = (pl.cdiv(N, _BLK),)
    dov = do.reshape(-1, 1)[:N]
    in_specs = [_row_spec(), _row_spec(dim=1)]
    args = [x, dov]
    flags = []
    for extra, dim in ((addto, D), (s, 1), (sel, 1)):
        if extra is not None:
            in_specs.append(_row_spec(dim=dim))
            args.append(extra if dim == D else extra.reshape(N, 1))
        flags.append(extra is not None)
    has_add, has_s, has_sel = flags

    def body(*refs):
        refs = list(refs)
        x_r, d_r = refs[0], refs[1]
        k = 2
        a_r = refs[k] if has_add else None
        k += has_add
        s_r = refs[k] if has_s else None
        k += has_s
        sl_r = refs[k] if has_sel else None
        o_r = refs[-1]
        out = x_r[...]
        if a_r is not None:
            out = out + a_r[...]
        out = out * lax.rsqrt(jnp.maximum(d_r[...], 1.0))
        if s_r is not None:
            out = out * s_r[...]
        if sl_r is not None:
            out = out * sl_r[...]
        o_r[...] = out

    return pl.pallas_call(
        body, grid=grid, in_specs=in_specs,
        out_specs=_row_spec(),
        out_shape=jax.ShapeDtypeStruct((N, D), jnp.float32),
    )(*args)


def kernel(x, edge_index, We, be, W0, b0, W1, b1, Wb, bb, Wd0, bd0, Wd1, bd1,
           Wp0, bp0, Wp1, bp1):
    src = edge_index[0]
    dst = edge_index[1]
    NA = 10240
    zN = jnp.zeros((NA, D), jnp.float32)
    zd = jnp.zeros((NA, RW), jnp.float32)
    onesr = jnp.zeros((CH, RW), jnp.float32).at[:, 0].set(1.0)

    mp = _make_mp(N, NA)
    degmp = _make_degmp(NA)
    degF_k = _make_deg(NA)

    dF = degF_k(src, dst, onesr, zd)
    doF = dF[0, :, 0]
    diF = dF[1, :, 0]

    xs = _tc_scale(x, doF)
    p = mp(xs, src, dst, zN)
    hs = _tc_layer((p[0], p[1]), diF, We, be, True)
    hs = _tc_scale(hs, doF)
    p = mp(hs, src, dst, zN)
    hid0, s0 = _tc_layer((p[0], p[1]), diF, W0, b0, True, proj=(Wp0, bp0))

    _, nids0 = lax.top_k(s0[:, 0], KS[0])
    sel1 = jnp.zeros((N,), jnp.float32).at[nids0].set(1.0)
    ind1 = sel1[:, None] * jnp.ones((1, D), jnp.float32)
    d1 = degmp(ind1, src, dst, zN)
    do1 = d1[0, :, 0]
    di1 = d1[1, :, 0]

    t1 = _tc_scale(hid0, do1, s=s0[:, 0], sel=sel1)
    p = mp(t1, src, dst, zN)
    hid1, s1 = _tc_layer((p[0], p[1]), di1, W1, b1, True, sel=sel1,
                         proj=(Wp1, bp1))

    _, nids1 = lax.top_k(s1[:, 0], KS[1])
    sel2 = jnp.zeros((N,), jnp.float32).at[nids1].set(1.0)
    ind2 = sel2[:, None] * jnp.ones((1, D), jnp.float32)
    d2 = degmp(ind2, src, dst, zN)
    do2 = d2[0, :, 0]
    di2 = d2[1, :, 0]

    t2 = _tc_scale(hid1, do2, s=s1[:, 0], sel=sel2)
    p = mp(t2, src, dst, zN)
    hb = _tc_layer((p[0], p[1]), di2, Wb, bb, True, sel=sel2)

    td0 = _tc_scale(hb, do1, sel=sel1, addto=hid1)
    p = mp(td0, src, dst, zN)
    hd0 = _tc_layer((p[0], p[1]), di1, Wd0, bd0, True, sel=sel1)

    td1 = _tc_scale(hd0, doF, addto=hid0)
    p = mp(td1, src, dst, zN)
    return _tc_layer((p[0], p[1]), diF, Wd1, bd1, False)


# speedup vs baseline: 9.5753x; 1.1920x over previous
import functools
import jax
import jax.numpy as jnp
from jax import lax
from jax.experimental import pallas as pl
from jax.experimental.pallas import tpu as pltpu
from jax.experimental.pallas import tpu_sc as plsc

N = 10000
E = 320000
D = 128
KS = (5000, 2500)

NC = 2
NS = 16
NW = NC * NS
CH = 128
NCHK = E // CH
NA = 10240
RW = 128
RB = 2

_MESH = plsc.VectorSubcoreMesh(core_axis_name="c", subcore_axis_name="s")


def _copy4(src_ref, dst_ref, base, rows):
    q = rows // 4

    def piece(j, _):
        off = base + j * q
        pltpu.sync_copy(src_ref.at[pl.ds(off, q)], dst_ref.at[pl.ds(off, q)])
        return 0

    lax.fori_loop(0, 4, piece, 0)


def _scan_ring(tab, gf, sf, acc, start, stride, nk, extra_pred, extra_c,
               sbufs, dbufs, rbufs, sgs, sss):
    def issue_g(b, c):
        pltpu.sync_copy(gf.at[pl.ds(c * CH, CH)], sbufs[b])
        return pltpu.async_copy(tab.at[sbufs[b]], rbufs[b], sgs[b])

    def issue_s(b, c, gdesc):
        gdesc.wait()
        pltpu.sync_copy(sf.at[pl.ds(c * CH, CH)], dbufs[b])
        return pltpu.async_copy(rbufs[b], acc.at[dbufs[b]], sss[b], add=True)

    def wait_s(b):
        pltpu.make_async_copy(rbufs[b], acc.at[dbufs[b]], sss[b]).wait()

    def cid_of(r, b):
        return start + (r * RB + b) * stride

    gd0 = [issue_g(b, cid_of(0, b)) for b in range(RB)]
    for b in range(RB):
        issue_s(b, cid_of(0, b), gd0[b])

    def round_fn(r, _):
        gd = []
        for b in range(RB):
            wait_s(b)
            gd.append(issue_g(b, cid_of(r, b)))
        for b in range(RB):
            issue_s(b, cid_of(r, b), gd[b])
        return 0

    lax.fori_loop(1, nk // RB, round_fn, 0)

    @pl.when(extra_pred)
    def _():
        wait_s(0)
        issue_s(0, extra_c, issue_g(0, extra_c))

    for b in range(RB):
        wait_s(b)


def _mp_body(tab, srcf, dstf, zer, out, acc, sv0, sv1, dv0, dv1,
             r0, r1, g0, g1, s0, s1, *, n_acc):
    cid = lax.axis_index("c")
    sid = lax.axis_index("s")
    wid = sid * NC + cid
    rpt = n_acc // NS
    _copy4(zer, acc, sid * rpt, rpt)
    plsc.subcore_barrier()
    _scan_ring(tab, srcf, dstf, acc, wid, NW, NCHK // NW,
               wid < NCHK % NW, wid + (NCHK // NW) * NW,
               (sv0, sv1), (dv0, dv1), (r0, r1), (g0, g1), (s0, s1))
    plsc.subcore_barrier()
    _copy4(acc, out.at[cid], sid * rpt, rpt)


def _make_mp(n_acc):
    body = functools.partial(_mp_body, n_acc=n_acc)
    return pl.kernel(
        body,
        out_type=jax.ShapeDtypeStruct((NC, n_acc, D), jnp.float32),
        mesh=_MESH,
        scratch_types=(
            [pltpu.VMEM_SHARED((n_acc, D), jnp.float32)]
            + [pltpu.VMEM((CH,), jnp.int32) for _ in range(4)]
            + [pltpu.VMEM((CH, D), jnp.float32) for _ in range(2)]
            + [pltpu.SemaphoreType.DMA for _ in range(4)]
        ),
    )


def _degmp_body(ind, srcf, dstf, zer, out, acc, sv0, sv1, dv0, dv1,
                r0, r1, g0, g1, s0, s1, *, n_acc):
    cid = lax.axis_index("c")
    sid = lax.axis_index("s")
    rpt = n_acc // NS
    _copy4(zer, acc, sid * rpt, rpt)
    plsc.subcore_barrier()
    bufs = ((sv0, sv1), (dv0, dv1), (r0, r1), (g0, g1), (s0, s1))

    @pl.when(cid == 0)
    def _():
        _scan_ring(ind, dstf, srcf, acc, sid, NS, NCHK // NS,
                   sid < NCHK % NS, sid + (NCHK // NS) * NS, *bufs)

    @pl.when(cid == 1)
    def _():
        _scan_ring(ind, srcf, dstf, acc, sid, NS, NCHK // NS,
                   sid < NCHK % NS, sid + (NCHK // NS) * NS, *bufs)

    plsc.subcore_barrier()
    _copy4(acc, out.at[cid], sid * rpt, rpt)


def _make_degmp(n_acc):
    body = functools.partial(_degmp_body, n_acc=n_acc)
    return pl.kernel(
        body,
        out_type=jax.ShapeDtypeStruct((NC, n_acc, RW), jnp.float32),
        mesh=_MESH,
        scratch_types=(
            [pltpu.VMEM_SHARED((n_acc, RW), jnp.float32)]
            + [pltpu.VMEM((CH,), jnp.int32) for _ in range(4)]
            + [pltpu.VMEM((CH, RW), jnp.float32) for _ in range(2)]
            + [pltpu.SemaphoreType.DMA for _ in range(4)]
        ),
    )


def _deg_body(srcf, dstf, onesr_h, zd_h, out, acc, idx_v, ones_v, sem,
              *, n_deg):
    cid = lax.axis_index("c")
    sid = lax.axis_index("s")
    rpt = n_deg // NS
    pltpu.sync_copy(onesr_h, ones_v)
    pltpu.sync_copy(zd_h.at[pl.ds(sid * rpt, rpt)],
                    acc.at[pl.ds(sid * rpt, rpt)])
    plsc.subcore_barrier()

    def scan_all(ef):
        def chunk(g, _):
            c = sid + g * NS
            pltpu.sync_copy(ef.at[pl.ds(c * CH, CH)], idx_v)
            pltpu.sync_copy(ones_v, acc.at[idx_v], add=True)
            return 0

        lax.fori_loop(0, NCHK // NS, chunk, 0)

        @pl.when(sid < NCHK % NS)
        def _():
            chunk(NCHK // NS, 0)

    @pl.when(cid == 0)
    def _():
        scan_all(srcf)

    @pl.when(cid == 1)
    def _():
        scan_all(dstf)

    plsc.subcore_barrier()
    pltpu.sync_copy(acc.at[pl.ds(sid * rpt, rpt)],
                    out.at[cid, pl.ds(sid * rpt, rpt)])


def _make_deg(n_deg):
    body = functools.partial(_deg_body, n_deg=n_deg)
    return pl.kernel(
        body,
        out_type=jax.ShapeDtypeStruct((NC, n_deg, RW), jnp.float32),
        mesh=_MESH,
        scratch_types=[
            pltpu.VMEM_SHARED((n_deg, RW), jnp.float32),
            pltpu.VMEM((CH,), jnp.int32),
            pltpu.VMEM((CH, RW), jnp.float32),
            pltpu.SemaphoreType.DMA,
        ],
    )


_BLK = 512


def _row_spec(blk=_BLK, dim=D):
    return pl.BlockSpec((blk, dim), lambda i: (i, 0))


def _one_spec(dim):
    return pl.BlockSpec((1, dim) if dim > 1 else (1, 1), lambda i: (0, 0))


def _layer_body(p0, p1, di, w, b, sel, o, *, relu):
    ni = lax.rsqrt(jnp.maximum(di[...], 1.0))
    agg = (p0[...] + p1[...]) * ni
    out = jnp.dot(agg, w[...], preferred_element_type=jnp.float32) + b[...]
    if relu:
        out = jnp.maximum(out, 0.0)
    if sel is not None:
        out = out * sel[...]
    o[...] = out


def _layer_proj_body(p0, p1, di, w, b, sel, wp, bp, o, sc, *, relu):
    ni = lax.rsqrt(jnp.maximum(di[...], 1.0))
    agg = (p0[...] + p1[...]) * ni
    out = jnp.dot(agg, w[...], preferred_element_type=jnp.float32) + b[...]
    if relu:
        out = jnp.maximum(out, 0.0)
    if sel is not None:
        out = out * sel[...]
    o[...] = out
    sv = jax.nn.sigmoid(
        jnp.dot(out, wp[...], preferred_element_type=jnp.float32) + bp[...])
    if sel is not None:
        sv = jnp.where(sel[...] > 0.0, sv, -1e30)
    sc[...] = sv


def _tc_layer(p, di, W, b, relu, sel=None, proj=None):
    grid = (pl.cdiv(N, _BLK),)
    div = di.reshape(-1, 1)[:N]
    in_specs = [_row_spec(), _row_spec(), _row_spec(dim=1),
                pl.BlockSpec((D, D), lambda i: (0, 0)), _one_spec(D)]
    args = [p[0], p[1], div, W, b.reshape(1, D)]
    has_sel = sel is not None
    if has_sel:
        in_specs.append(_row_spec(dim=1))
        args.append(sel.reshape(N, 1))
    if proj is None:
        def body(*refs):
            if has_sel:
                p0, p1, dr, w, b_, sl, o = refs
            else:
                (p0, p1, dr, w, b_, o), sl = refs, None
            _layer_body(p0, p1, dr, w, b_, sl, o, relu=relu)
        return pl.pallas_call(
            body, grid=grid, in_specs=in_specs,
            out_specs=_row_spec(),
            out_shape=jax.ShapeDtypeStruct((N, D), jnp.float32),
        )(*args)
    Wp, bp = proj

    def bodyp(*refs):
        if has_sel:
            p0, p1, dr, w, b_, sl, wp, bp_, o, sc = refs
        else:
            (p0, p1, dr, w, b_, wp, bp_, o, sc), sl = refs, None
        _layer_proj_body(p0, p1, dr, w, b_, sl, wp, bp_, o, sc, relu=relu)
    return pl.pallas_call(
        bodyp, grid=grid,
        in_specs=in_specs + [pl.BlockSpec((D, 1), lambda i: (0, 0)),
                             _one_spec(1)],
        out_specs=[_row_spec(), _row_spec(dim=1)],
        out_shape=[jax.ShapeDtypeStruct((N, D), jnp.float32),
                   jax.ShapeDtypeStruct((N, 1), jnp.float32)],
    )(*args, Wp, bp.reshape(1, 1))


def _tc_scale(x, do, s=None, sel=None, addto=None):
    grid = (pl.cdiv(N, _BLK),)
    dov = do.reshape(-1, 1)[:N]
    in_specs = [_row_spec(), _row_spec(dim=1)]
    args = [x, dov]
    flags = []
    for extra, dim in ((addto, D), (s, 1), (sel, 1)):
        if extra is not None:
            in_specs.append(_row_spec(dim=dim))
            args.append(extra if dim == D else extra.reshape(N, 1))
        flags.append(extra is not None)
    has_add, has_s, has_sel = flags

    def body(*refs):
        x_r, d_r = refs[0], refs[1]
        k = 2
        a_r = refs[k] if has_add else None
        k += has_add
        s_r = refs[k] if has_s else None
        k += has_s
        sl_r = refs[k] if has_sel else None
        o_r = refs[-1]
        out = x_r[...]
        if a_r is not None:
            out = out + a_r[...]
        out = out * lax.rsqrt(jnp.maximum(d_r[...], 1.0))
        if s_r is not None:
            out = out * s_r[...]
        if sl_r is not None:
            out = out * sl_r[...]
        o_r[...] = out

    return pl.pallas_call(
        body, grid=grid, in_specs=in_specs,
        out_specs=_row_spec(),
        out_shape=jax.ShapeDtypeStruct((N, D), jnp.float32),
    )(*args)


def kernel(x, edge_index, We, be, W0, b0, W1, b1, Wb, bb, Wd0, bd0, Wd1, bd1,
           Wp0, bp0, Wp1, bp1):
    src = edge_index[0]
    dst = edge_index[1]
    zN = jnp.zeros((NA, D), jnp.float32)
    zd = jnp.zeros((NA, RW), jnp.float32)
    onesr = jnp.zeros((CH, RW), jnp.float32).at[:, 0].set(1.0)

    mp = _make_mp(NA)
    degmp = _make_degmp(NA)
    deg_full = _make_deg(NA)

    dF = deg_full(src, dst, onesr, zd)
    doF = dF[0, :, 0]
    diF = dF[1, :, 0]

    xs = _tc_scale(x, doF)
    p = mp(xs, src, dst, zN)
    hs = _tc_layer((p[0], p[1]), diF, We, be, True)
    hs = _tc_scale(hs, doF)
    p = mp(hs, src, dst, zN)
    hid0, s0 = _tc_layer((p[0], p[1]), diF, W0, b0, True, proj=(Wp0, bp0))

    _, nids0 = lax.top_k(s0[:, 0], KS[0])
    sel1 = jnp.zeros((N,), jnp.float32).at[nids0].set(1.0)
    ind1 = sel1[:, None] * jnp.ones((1, RW), jnp.float32)
    d1 = degmp(ind1, src, dst, zd)
    do1 = d1[0, :, 0]
    di1 = d1[1, :, 0]

    t1 = _tc_scale(hid0, do1, s=s0[:, 0], sel=sel1)
    p = mp(t1, src, dst, zN)
    hid1, s1 = _tc_layer((p[0], p[1]), di1, W1, b1, True, sel=sel1,
                         proj=(Wp1, bp1))

    _, nids1 = lax.top_k(s1[:, 0], KS[1])
    sel2 = jnp.zeros((N,), jnp.float32).at[nids1].set(1.0)
    ind2 = sel2[:, None] * jnp.ones((1, RW), jnp.float32)
    d2 = degmp(ind2, src, dst, zd)
    do2 = d2[0, :, 0]
    di2 = d2[1, :, 0]

    t2 = _tc_scale(hid1, do2, s=s1[:, 0], sel=sel2)
    p = mp(t2, src, dst, zN)
    hb = _tc_layer((p[0], p[1]), di2, Wb, bb, True, sel=sel2)

    td0 = _tc_scale(hb, do1, sel=sel1, addto=hid1)
    p = mp(td0, src, dst, zN)
    hd0 = _tc_layer((p[0], p[1]), di1, Wd0, bd0, True, sel=sel1)

    td1 = _tc_scale(hd0, doF, addto=hid0)
    p = mp(td1, src, dst, zN)
    return _tc_layer((p[0], p[1]), diF, Wd1, bd1, False)


# speedup vs baseline: 9.9075x; 1.0347x over previous
import functools
import jax
import jax.numpy as jnp
from jax import lax
from jax.experimental import pallas as pl
from jax.experimental.pallas import tpu as pltpu
from jax.experimental.pallas import tpu_sc as plsc

N = 10000
E = 320000
D = 128
KS = (5000, 2500)

NC = 2
NS = 16
NW = NC * NS
CH = 128
NCHK = E // CH
NA = 10240
RW = 128
RB = 2

_MESH = plsc.VectorSubcoreMesh(core_axis_name="c", subcore_axis_name="s")


def _copy4(src_ref, dst_ref, base, rows):
    q = rows // 4

    def piece(j, _):
        off = base + j * q
        pltpu.sync_copy(src_ref.at[pl.ds(off, q)], dst_ref.at[pl.ds(off, q)])
        return 0

    lax.fori_loop(0, 4, piece, 0)


def _scan_ring(tab, gf, sf, acc, start, stride, nk, extra_pred, extra_c,
               sbufs, dbufs, rbufs, sgs, sss):
    def issue_g(b, c):
        pltpu.sync_copy(gf.at[pl.ds(c * CH, CH)], sbufs[b])
        return pltpu.async_copy(tab.at[sbufs[b]], rbufs[b], sgs[b])

    def issue_s(b, c, gdesc):
        gdesc.wait()
        pltpu.sync_copy(sf.at[pl.ds(c * CH, CH)], dbufs[b])
        return pltpu.async_copy(rbufs[b], acc.at[dbufs[b]], sss[b], add=True)

    def wait_s(b):
        pltpu.make_async_copy(rbufs[b], acc.at[dbufs[b]], sss[b]).wait()

    def cid_of(r, b):
        return start + (r * RB + b) * stride

    gd0 = [issue_g(b, cid_of(0, b)) for b in range(RB)]
    for b in range(RB):
        issue_s(b, cid_of(0, b), gd0[b])

    def round_fn(r, _):
        gd = []
        for b in range(RB):
            wait_s(b)
            gd.append(issue_g(b, cid_of(r, b)))
        for b in range(RB):
            issue_s(b, cid_of(r, b), gd[b])
        return 0

    lax.fori_loop(1, nk // RB, round_fn, 0)

    @pl.when(extra_pred)
    def _():
        wait_s(0)
        issue_s(0, extra_c, issue_g(0, extra_c))

    for b in range(RB):
        wait_s(b)


def _mp_body(tab, srcf, dstf, zer, out, acc, sv0, sv1, dv0, dv1,
             r0, r1, g0, g1, s0, s1, *, n_acc):
    cid = lax.axis_index("c")
    sid = lax.axis_index("s")
    wid = sid * NC + cid
    rpt = n_acc // NS
    _copy4(zer, acc, sid * rpt, rpt)
    plsc.subcore_barrier()
    _scan_ring(tab, srcf, dstf, acc, wid, NW, NCHK // NW,
               wid < NCHK % NW, wid + (NCHK // NW) * NW,
               (sv0, sv1), (dv0, dv1), (r0, r1), (g0, g1), (s0, s1))
    plsc.subcore_barrier()
    _copy4(acc, out.at[cid], sid * rpt, rpt)


def _make_mp(n_acc):
    body = functools.partial(_mp_body, n_acc=n_acc)
    return pl.kernel(
        body,
        out_type=jax.ShapeDtypeStruct((NC, n_acc, D), jnp.float32),
        mesh=_MESH,
        scratch_types=(
            [pltpu.VMEM_SHARED((n_acc, D), jnp.float32)]
            + [pltpu.VMEM((CH,), jnp.int32) for _ in range(4)]
            + [pltpu.VMEM((CH, D), jnp.float32) for _ in range(2)]
            + [pltpu.SemaphoreType.DMA for _ in range(4)]
        ),
    )


def _degmp_body(ind, srcf, dstf, zer, out, acc, sv0, sv1, dv0, dv1,
                r0, r1, g0, g1, s0, s1, *, n_acc):
    cid = lax.axis_index("c")
    sid = lax.axis_index("s")
    rpt = n_acc // NS
    _copy4(zer, acc, sid * rpt, rpt)
    plsc.subcore_barrier()
    bufs = ((sv0, sv1), (dv0, dv1), (r0, r1), (g0, g1), (s0, s1))

    @pl.when(cid == 0)
    def _():
        _scan_ring(ind, dstf, srcf, acc, sid, NS, NCHK // NS,
                   sid < NCHK % NS, sid + (NCHK // NS) * NS, *bufs)

    @pl.when(cid == 1)
    def _():
        _scan_ring(ind, srcf, dstf, acc, sid, NS, NCHK // NS,
                   sid < NCHK % NS, sid + (NCHK // NS) * NS, *bufs)

    plsc.subcore_barrier()
    _copy4(acc, out.at[cid], sid * rpt, rpt)


def _make_degmp(n_acc):
    body = functools.partial(_degmp_body, n_acc=n_acc)
    return pl.kernel(
        body,
        out_type=jax.ShapeDtypeStruct((NC, n_acc, RW), jnp.float32),
        mesh=_MESH,
        scratch_types=(
            [pltpu.VMEM_SHARED((n_acc, RW), jnp.float32)]
            + [pltpu.VMEM((CH,), jnp.int32) for _ in range(4)]
            + [pltpu.VMEM((CH, RW), jnp.float32) for _ in range(2)]
            + [pltpu.SemaphoreType.DMA for _ in range(4)]
        ),
    )


def _deg_body(srcf, dstf, onesr_h, zd_h, out, acc, iv0, iv1, ones_v,
              s0, s1, *, n_deg):
    cid = lax.axis_index("c")
    sid = lax.axis_index("s")
    rpt = n_deg // NS
    pltpu.sync_copy(onesr_h, ones_v)
    _copy4(zd_h, acc, sid * rpt, rpt)
    plsc.subcore_barrier()

    def scan_all(ef):
        ivs = (iv0, iv1)
        sss = (s0, s1)

        def issue(b, c):
            pltpu.sync_copy(ef.at[pl.ds(c * CH, CH)], ivs[b])
            return pltpu.async_copy(ones_v, acc.at[ivs[b]], sss[b], add=True)

        def wait_s(b):
            pltpu.make_async_copy(ones_v, acc.at[ivs[b]], sss[b]).wait()

        for b in range(2):
            issue(b, sid + b * NS)

        def round_fn(r, _):
            for b in range(2):
                wait_s(b)
                issue(b, sid + (2 * r + b) * NS)
            return 0

        lax.fori_loop(1, (NCHK // NS) // 2, round_fn, 0)

        @pl.when(sid < NCHK % NS)
        def _():
            wait_s(0)
            issue(0, sid + (NCHK // NS) * NS)

        for b in range(2):
            wait_s(b)

    @pl.when(cid == 0)
    def _():
        scan_all(srcf)

    @pl.when(cid == 1)
    def _():
        scan_all(dstf)

    plsc.subcore_barrier()
    _copy4(acc, out.at[cid], sid * rpt, rpt)


def _make_deg(n_deg):
    body = functools.partial(_deg_body, n_deg=n_deg)
    return pl.kernel(
        body,
        out_type=jax.ShapeDtypeStruct((NC, n_deg, RW), jnp.float32),
        mesh=_MESH,
        scratch_types=[
            pltpu.VMEM_SHARED((n_deg, RW), jnp.float32),
            pltpu.VMEM((CH,), jnp.int32),
            pltpu.VMEM((CH,), jnp.int32),
            pltpu.VMEM((CH, RW), jnp.float32),
            pltpu.SemaphoreType.DMA,
            pltpu.SemaphoreType.DMA,
        ],
    )


_BLK = 512


def _row_spec(blk=_BLK, dim=D):
    return pl.BlockSpec((blk, dim), lambda i: (i, 0))


def _one_spec(dim):
    return pl.BlockSpec((1, dim) if dim > 1 else (1, 1), lambda i: (0, 0))


def _layer_body(p0, p1, di, w, b, sel, o, *, relu):
    ni = lax.rsqrt(jnp.maximum(di[...], 1.0))
    agg = (p0[...] + p1[...]) * ni
    out = jnp.dot(agg, w[...], preferred_element_type=jnp.float32) + b[...]
    if relu:
        out = jnp.maximum(out, 0.0)
    if sel is not None:
        out = out * sel[...]
    o[...] = out


def _layer_proj_body(p0, p1, di, w, b, sel, wp, bp, o, sc, *, relu):
    ni = lax.rsqrt(jnp.maximum(di[...], 1.0))
    agg = (p0[...] + p1[...]) * ni
    out = jnp.dot(agg, w[...], preferred_element_type=jnp.float32) + b[...]
    if relu:
        out = jnp.maximum(out, 0.0)
    if sel is not None:
        out = out * sel[...]
    o[...] = out
    sv = jax.nn.sigmoid(
        jnp.dot(out, wp[...], preferred_element_type=jnp.float32) + bp[...])
    if sel is not None:
        sv = jnp.where(sel[...] > 0.0, sv, -1e30)
    sc[...] = sv


def _tc_layer(p, di, W, b, relu, sel=None, proj=None):
    grid = (pl.cdiv(N, _BLK),)
    div = di.reshape(-1, 1)[:N]
    in_specs = [_row_spec(), _row_spec(), _row_spec(dim=1),
                pl.BlockSpec((D, D), lambda i: (0, 0)), _one_spec(D)]
    args = [p[0], p[1], div, W, b.reshape(1, D)]
    has_sel = sel is not None
    if has_sel:
        in_specs.append(_row_spec(dim=1))
        args.append(sel.reshape(N, 1))
    if proj is None:
        def body(*refs):
            if has_sel:
                p0, p1, dr, w, b_, sl, o = refs
            else:
                (p0, p1, dr, w, b_, o), sl = refs, None
            _layer_body(p0, p1, dr, w, b_, sl, o, relu=relu)
        return pl.pallas_call(
            body, grid=grid, in_specs=in_specs,
            out_specs=_row_spec(),
            out_shape=jax.ShapeDtypeStruct((N, D), jnp.float32),
        )(*args)
    Wp, bp = proj

    def bodyp(*refs):
        if has_sel:
            p0, p1, dr, w, b_, sl, wp, bp_, o, sc = refs
        else:
            (p0, p1, dr, w, b_, wp, bp_, o, sc), sl = refs, None
        _layer_proj_body(p0, p1, dr, w, b_, sl, wp, bp_, o, sc, relu=relu)
    return pl.pallas_call(
        bodyp, grid=grid,
        in_specs=in_specs + [pl.BlockSpec((D, 1), lambda i: (0, 0)),
                             _one_spec(1)],
        out_specs=[_row_spec(), _row_spec(dim=1)],
        out_shape=[jax.ShapeDtypeStruct((N, D), jnp.float32),
                   jax.ShapeDtypeStruct((N, 1), jnp.float32)],
    )(*args, Wp, bp.reshape(1, 1))


def _tc_scale(x, do, s=None, sel=None, addto=None):
    grid = (pl.cdiv(N, _BLK),)
    dov = do.reshape(-1, 1)[:N]
    in_specs = [_row_spec(), _row_spec(dim=1)]
    args = [x, dov]
    flags = []
    for extra, dim in ((addto, D), (s, 1), (sel, 1)):
        if extra is not None:
            in_specs.append(_row_spec(dim=dim))
            args.append(extra if dim == D else extra.reshape(N, 1))
        flags.append(extra is not None)
    has_add, has_s, has_sel = flags

    def body(*refs):
        x_r, d_r = refs[0], refs[1]
        k = 2
        a_r = refs[k] if has_add else None
        k += has_add
        s_r = refs[k] if has_s else None
        k += has_s
        sl_r = refs[k] if has_sel else None
        o_r = refs[-1]
        out = x_r[...]
        if a_r is not None:
            out = out + a_r[...]
        out = out * lax.rsqrt(jnp.maximum(d_r[...], 1.0))
        if s_r is not None:
            out = out * s_r[...]
        if sl_r is not None:
            out = out * sl_r[...]
        o_r[...] = out

    return pl.pallas_call(
        body, grid=grid, in_specs=in_specs,
        out_specs=_row_spec(),
        out_shape=jax.ShapeDtypeStruct((N, D), jnp.float32),
    )(*args)


def kernel(x, edge_index, We, be, W0, b0, W1, b1, Wb, bb, Wd0, bd0, Wd1, bd1,
           Wp0, bp0, Wp1, bp1):
    src = edge_index[0]
    dst = edge_index[1]
    zN = jnp.zeros((NA, D), jnp.float32)
    zd = jnp.zeros((NA, RW), jnp.float32)
    onesr = jnp.zeros((CH, RW), jnp.float32).at[:, 0].set(1.0)

    mp = _make_mp(NA)
    degmp = _make_degmp(NA)
    deg_full = _make_deg(NA)

    dF = deg_full(src, dst, onesr, zd)
    doF = dF[0, :, 0]
    diF = dF[1, :, 0]

    xs = _tc_scale(x, doF)
    p = mp(xs, src, dst, zN)
    hs = _tc_layer((p[0], p[1]), diF, We, be, True)
    hs = _tc_scale(hs, doF)
    p = mp(hs, src, dst, zN)
    hid0, s0 = _tc_layer((p[0], p[1]), diF, W0, b0, True, proj=(Wp0, bp0))

    _, nids0 = lax.top_k(s0[:, 0], KS[0])
    sel1 = jnp.zeros((N,), jnp.float32).at[nids0].set(1.0)
    ind1 = sel1[:, None] * jnp.ones((1, RW), jnp.float32)
    d1 = degmp(ind1, src, dst, zd)
    do1 = d1[0, :, 0]
    di1 = d1[1, :, 0]

    t1 = _tc_scale(hid0, do1, s=s0[:, 0], sel=sel1)
    p = mp(t1, src, dst, zN)
    hid1, s1 = _tc_layer((p[0], p[1]), di1, W1, b1, True, sel=sel1,
                         proj=(Wp1, bp1))

    _, nids1 = lax.top_k(s1[:, 0], KS[1])
    sel2 = jnp.zeros((N,), jnp.float32).at[nids1].set(1.0)
    ind2 = sel2[:, None] * jnp.ones((1, RW), jnp.float32)
    d2 = degmp(ind2, src, dst, zd)
    do2 = d2[0, :, 0]
    di2 = d2[1, :, 0]

    t2 = _tc_scale(hid1, do2, s=s1[:, 0], sel=sel2)
    p = mp(t2, src, dst, zN)
    hb = _tc_layer((p[0], p[1]), di2, Wb, bb, True, sel=sel2)

    td0 = _tc_scale(hb, do1, sel=sel1, addto=hid1)
    p = mp(td0, src, dst, zN)
    hd0 = _tc_layer((p[0], p[1]), di1, Wd0, bd0, True, sel=sel1)

    td1 = _tc_scale(hd0, doF, addto=hid0)
    p = mp(td1, src, dst, zN)
    return _tc_layer((p[0], p[1]), diF, Wd1, bd1, False)
